# baseline (speedup 1.0000x reference)
import jax
import jax.numpy as jnp
from jax import lax

MAT = 8192
N = MAT * MAT


def kernel(x, w, rows, cols):
    flat = rows * MAT + cols
    fs, ws = lax.sort_key_val(flat, w, is_stable=False)
    is_last = jnp.concatenate([fs[:-1] != fs[1:], jnp.array([True])])
    target = jnp.where(is_last, fs, N)
    out = jnp.zeros((N + 1,), w.dtype).at[target].set(ws, unique_indices=True)
    return out[:N].reshape(MAT, MAT)


# speedup vs baseline: 1.1814x; 1.1814x over previous
import functools

import jax
import jax.numpy as jnp
from jax import lax
from jax.experimental import pallas as pl
from jax.experimental.pallas import tpu as pltpu
from jax.experimental.pallas import tpu_sc as plsc

MAT = 8192
N = MAT * MAT
NU = 1048576
PAD = NU
TOT = N + PAD

NC = 2
NS = 16
HALF = N // NC
ZSLAB = N // (NC * NS)
ZBUF = 32768
NZDMA = ZSLAB // ZBUF
CHUNK = NU // NS
SB = 8192
NSB = CHUNK // SB
ROWS = SB // 128


def _sc_body(tgt_hbm, ws_hbm, out_hbm, zbuf, idxb, valb, zsem, csem):
    c = lax.axis_index("c")
    s = lax.axis_index("s")
    lane = lax.iota(jnp.int32, 16)

    def _zfill(i, _):
        zbuf[pl.ds(i * 16, 16)] = jnp.zeros((16,), jnp.float32)
        return 0

    lax.fori_loop(0, ZBUF // 16, _zfill, 0)

    zb = c * HALF + s * ZSLAB
    for g in range(NZDMA // 4):
        descs = [
            pltpu.async_copy(
                zbuf, out_hbm.at[pl.ds(zb + (g * 4 + k) * ZBUF, ZBUF)], zsem
            )
            for k in range(4)
        ]
        for d in descs:
            d.wait()

    plsc.subcore_barrier()

    lo = c * HALF
    hi = lo + HALF

    for sb in range(NSB):
        rbase = s * (CHUNK // 128) + sb * ROWS
        pltpu.sync_copy(tgt_hbm.at[pl.ds(rbase, ROWS)], idxb)
        pltpu.sync_copy(ws_hbm.at[pl.ds(rbase, ROWS)], valb)

        ebase = s * CHUNK + sb * SB

        def _mask(t, _):
            j = t // 8
            kk = t % 8
            cur = idxb[j, pl.ds(kk * 16, 16)]
            owned = jnp.logical_and(cur >= lo, cur < hi)
            dumpv = (N + ebase + t * 16) + lane
            idxb[j, pl.ds(kk * 16, 16)] = jnp.where(owned, cur, dumpv)
            return 0

        lax.fori_loop(0, SB // 16, _mask, 0)

        for g in range(ROWS // 8):
            descs = [
                pltpu.async_copy(
                    valb.at[g * 8 + k],
                    out_hbm.at[idxb.at[g * 8 + k]],
                    csem,
                )
                for k in range(8)
            ]
            for d in descs:
                d.wait()


@jax.jit
def _sc_scatter(tgt2d, ws2d):
    mesh = plsc.VectorSubcoreMesh(core_axis_name="c", subcore_axis_name="s")
    return pl.kernel(
        _sc_body,
        out_type=jax.ShapeDtypeStruct((TOT,), jnp.float32),
        mesh=mesh,
        scratch_types=[
            pltpu.VMEM((ZBUF,), jnp.float32),
            pltpu.VMEM((ROWS, 128), jnp.int32),
            pltpu.VMEM((ROWS, 128), jnp.float32),
            pltpu.SemaphoreType.DMA,
            pltpu.SemaphoreType.DMA,
        ],
    )(tgt2d, ws2d)


def kernel(x, w, rows, cols):
    flat = rows * MAT + cols
    fs, ws = lax.sort_key_val(flat, w, is_stable=False)
    is_last = jnp.concatenate([fs[:-1] != fs[1:], jnp.full((1,), True)])
    tgt = jnp.where(is_last, fs, -8)
    out1d = _sc_scatter(tgt.reshape(NU // 128, 128), ws.reshape(NU // 128, 128))
    return out1d[:N].reshape(MAT, MAT)


# speedup vs baseline: 1.1820x; 1.0005x over previous
import functools

import jax
import jax.numpy as jnp
from jax import lax
from jax.experimental import pallas as pl
from jax.experimental.pallas import tpu as pltpu
from jax.experimental.pallas import tpu_sc as plsc

MAT = 8192
N = MAT * MAT
NU = 1048576
PAD = NU
TOT = N + PAD

NC = 2
NS = 16
HALF = N // NC
ZSLAB = N // (NC * NS)
ZBUF = 32768
NZDMA = ZSLAB // ZBUF
CHUNK = NU // NS
SB = 8192
NSB = CHUNK // SB
ROWS = SB // 128


def _sc_body(tgt_hbm, ws_hbm, out_hbm, zbuf, idxb, valb, zsem, csem):
    c = lax.axis_index("c")
    s = lax.axis_index("s")
    lane = lax.iota(jnp.int32, 16)

    def _zfill(i, _):
        zbuf[pl.ds(i * 16, 16)] = jnp.zeros((16,), jnp.float32)
        return 0

    lax.fori_loop(0, ZBUF // 16, _zfill, 0)

    zb = c * HALF + s * ZSLAB
    for g in range(NZDMA // 4):
        descs = [
            pltpu.async_copy(
                zbuf, out_hbm.at[pl.ds(zb + (g * 4 + k) * ZBUF, ZBUF)], zsem
            )
            for k in range(4)
        ]
        for d in descs:
            d.wait()

    plsc.subcore_barrier()

    lo = c * HALF
    hi = lo + HALF

    for sb in range(NSB):
        ebase = s * CHUNK + sb * SB
        pltpu.sync_copy(tgt_hbm.at[pl.ds(ebase, SB)], idxb)
        pltpu.sync_copy(ws_hbm.at[pl.ds(ebase, SB)], valb)

        def _mask(t, _):
            cur = idxb[pl.ds(t * 16, 16)]
            owned = jnp.logical_and(cur >= lo, cur < hi)
            dumpv = (N + ebase + t * 16) + lane
            idxb[pl.ds(t * 16, 16)] = jnp.where(owned, cur, dumpv)
            return 0

        lax.fori_loop(0, SB // 16, _mask, 0)

        pltpu.async_copy(valb, out_hbm.at[idxb], csem).wait()


@jax.jit
def _sc_scatter(tgt2d, ws2d):
    mesh = plsc.VectorSubcoreMesh(core_axis_name="c", subcore_axis_name="s")
    return pl.kernel(
        _sc_body,
        out_type=jax.ShapeDtypeStruct((TOT,), jnp.float32),
        mesh=mesh,
        scratch_types=[
            pltpu.VMEM((ZBUF,), jnp.float32),
            pltpu.VMEM((SB,), jnp.int32),
            pltpu.VMEM((SB,), jnp.float32),
            pltpu.SemaphoreType.DMA,
            pltpu.SemaphoreType.DMA,
        ],
    )(tgt2d, ws2d)


def kernel(x, w, rows, cols):
    flat = rows * MAT + cols
    fs, ws = lax.sort_key_val(flat, w, is_stable=False)
    is_last = jnp.concatenate([fs[:-1] != fs[1:], jnp.full((1,), True)])
    tgt = jnp.where(is_last, fs, -8)
    out1d = _sc_scatter(tgt, ws)
    return out1d[:N].reshape(MAT, MAT)


# speedup vs baseline: 1.3960x; 1.1811x over previous
import functools

import jax
import jax.numpy as jnp
from jax import lax
from jax.experimental import pallas as pl
from jax.experimental.pallas import tpu as pltpu
from jax.experimental.pallas import tpu_sc as plsc

MAT = 8192
N = MAT * MAT
NU = 1048576
PAD = NU * 16
TOT = N + PAD

NC = 2
NS = 16
HALF = N // NC
ZSLAB = N // (NC * NS)
ZBUF = 32768
NZDMA = ZSLAB // ZBUF
CHUNK = NU // NS
SB = 8192
NSB = CHUNK // SB
ROWS = SB // 128


def _sc_body(tgt_hbm, ws_hbm, out_hbm, zbuf, idxb, valb, zsem, csem):
    c = lax.axis_index("c")
    s = lax.axis_index("s")
    lane16 = lax.iota(jnp.int32, 16) * 16

    def _zfill(i, _):
        zbuf[pl.ds(i * 16, 16)] = jnp.zeros((16,), jnp.float32)
        return 0

    lax.fori_loop(0, ZBUF // 16, _zfill, 0)

    zb = c * HALF + s * ZSLAB
    for g in range(NZDMA // 4):
        descs = [
            pltpu.async_copy(
                zbuf, out_hbm.at[pl.ds(zb + (g * 4 + k) * ZBUF, ZBUF)], zsem
            )
            for k in range(4)
        ]
        for d in descs:
            d.wait()

    plsc.subcore_barrier()

    lo = c * HALF
    hi = lo + HALF

    for sb in range(NSB):
        ebase = s * CHUNK + sb * SB
        pltpu.sync_copy(tgt_hbm.at[pl.ds(ebase, SB)], idxb)
        pltpu.sync_copy(ws_hbm.at[pl.ds(ebase, SB)], valb)

        def _mask(t, _):
            cur = idxb[pl.ds(t * 16, 16)]
            owned = jnp.logical_and(cur >= lo, cur < hi)
            dumpv = (N + (ebase + t * 16) * 16) + lane16
            idxb[pl.ds(t * 16, 16)] = jnp.where(owned, cur, dumpv)
            return 0

        lax.fori_loop(0, SB // 16, _mask, 0)

        pltpu.async_copy(valb, out_hbm.at[idxb], csem).wait()


@jax.jit
def _sc_scatter(tgt2d, ws2d):
    mesh = plsc.VectorSubcoreMesh(core_axis_name="c", subcore_axis_name="s")
    return pl.kernel(
        _sc_body,
        out_type=jax.ShapeDtypeStruct((TOT,), jnp.float32),
        mesh=mesh,
        scratch_types=[
            pltpu.VMEM((ZBUF,), jnp.float32),
            pltpu.VMEM((SB,), jnp.int32),
            pltpu.VMEM((SB,), jnp.float32),
            pltpu.SemaphoreType.DMA,
            pltpu.SemaphoreType.DMA,
        ],
    )(tgt2d, ws2d)


def kernel(x, w, rows, cols):
    flat = rows * MAT + cols
    fs, ws = lax.sort_key_val(flat, w, is_stable=False)
    is_last = jnp.concatenate([fs[:-1] != fs[1:], jnp.full((1,), True)])
    tgt = jnp.where(is_last, fs, -8)
    out1d = _sc_scatter(tgt, ws)
    return out1d[:N].reshape(MAT, MAT)
